# baseline (speedup 1.0000x reference)
import functools

import jax
import jax.numpy as jnp
from jax import lax
from jax.experimental import pallas as pl
from jax.experimental.pallas import tpu as pltpu
from jax.experimental.pallas import tpu_sc as plsc


def kernel(idx, mu_weight, sigma_weight):
    B = idx.shape[0]
    V, D = mu_weight.shape
    info = plsc.get_sparse_core_info()
    NC, NS, L = info.num_cores, info.num_subcores, info.num_lanes
    NW = NC * NS
    assert B % (L * NW) == 0 and D % L == 0 and V % 2 == 0
    bpw = B // NW

    mu2 = mu_weight.reshape(V // 2, 2 * D)
    sig2 = sigma_weight.reshape(V // 2, 2 * D)

    mesh = plsc.VectorSubcoreMesh(core_axis_name="c", subcore_axis_name="s")

    @functools.partial(
        pl.kernel,
        mesh=mesh,
        compiler_params=pltpu.CompilerParams(use_tc_tiling_on_sc=True),
        out_type=jax.ShapeDtypeStruct((2 * B, D), jnp.float32),
        scratch_types=[
            pltpu.VMEM((bpw,), jnp.int32),
            pltpu.VMEM((bpw,), jnp.int32),
            pltpu.VMEM((bpw, 2 * D), jnp.float32),
            pltpu.VMEM((bpw, 2 * D), jnp.float32),
            pltpu.VMEM((2 * bpw, D), jnp.float32),
            pltpu.SemaphoreType.DMA,
            pltpu.SemaphoreType.DMA,
        ],
    )
    def run(idx_hbm, mu_hbm, sig_hbm, out_hbm,
            idx_v, pair_v, mu_v, sig_v, out_v, sem_mu, sem_sig):
        wid = lax.axis_index("s") * NC + lax.axis_index("c")
        base = wid * bpw
        pltpu.sync_copy(idx_hbm.at[pl.ds(base, bpw)], idx_v)

        for i in range(bpw // L):
            rv = idx_v[pl.ds(i * L, L)]
            pair_v[pl.ds(i * L, L)] = rv >> 1

        mu_cp = pltpu.async_copy(mu_hbm.at[pair_v], mu_v, sem_mu)
        sig_cp = pltpu.async_copy(sig_hbm.at[pair_v], sig_v, sem_sig)
        mu_cp.wait()
        sig_cp.wait()

        for i in range(bpw // L):
            rv = idx_v[pl.ds(i * L, L)]
            for l in range(L):
                j = i * L + l
                off = (rv[l] & 1) * D
                for cb in range(D // L):
                    mv = mu_v[j, pl.ds(off + cb * L, L)]
                    out_v[2 * j, pl.ds(cb * L, L)] = mv
                for cb in range(D // L):
                    sv = sig_v[j, pl.ds(off + cb * L, L)]
                    out_v[2 * j + 1, pl.ds(cb * L, L)] = (
                        jnp.maximum(sv, 0.0) + jnp.exp(jnp.minimum(sv, 0.0)))

        pltpu.sync_copy(out_v, out_hbm.at[pl.ds(2 * base, 2 * bpw)])

    out2 = run(idx, mu2, sig2)
    return out2.reshape(B, 2 * D)
